# baseline (speedup 1.0000x reference)
import functools

import jax
import jax.numpy as jnp
from jax import lax
from jax.experimental import pallas as pl
from jax.experimental.pallas import tpu as pltpu
from jax.experimental.pallas import tpu_sc as plsc

L = 16
NS = 16
NC = 2
BH = 128
NB = BH // L
K = 64


def _rup(a, b):
    return -(-a // b) * b


SB = 4
D = 2


@functools.partial(jax.jit, static_argnums=(4, 5))
def _sc_segment(xT2, src_g, dst_g, w_g, n_out_pad, n_super):
    mesh = plsc.VectorSubcoreMesh(core_axis_name="c", subcore_axis_name="s")
    nz = n_out_pad // NS

    def body(xT2_hbm, src_hbm, dst_hbm, w_hbm, out_hbm,
             sv, dv, wv, rows, sbuf, zb_vm, acc_sh,
             gsems, ssems, esem):
        c = lax.axis_index("c")
        s = lax.axis_index("s")
        row0 = s * nz
        zv = jnp.zeros((L,), jnp.float32)
        for r in range(16):
            for j in range(NB):
                zb_vm[r, pl.ds(j * L, L)] = zv

        def zloop(g, carry):
            pltpu.sync_copy(zb_vm, acc_sh.at[pl.ds(row0 + g * 16, 16)])
            return carry
        lax.fori_loop(0, nz // 16, zloop, 0)
        plsc.subcore_barrier()

        pltpu.sync_copy(src_hbm.at[c, s, 0], sv.at[0])
        pltpu.sync_copy(dst_hbm.at[s, 0], dv.at[0])
        pltpu.sync_copy(w_hbm.at[s, 0], wv.at[0])

        def scale(sl, cch):
            b = cch % D
            for h in range(K // L):
                wrow = wv[sl, cch, pl.ds(h * L, L)]
                for e16 in range(L):
                    w1 = jnp.full((L,), wrow[e16])
                    e = h * L + e16
                    for j in range(NB):
                        sbuf[b, e, pl.ds(j * L, L)] = rows[b, e, pl.ds(j * L, L)] * w1

        def sloop(og, carry):
            sl = og % 2
            nxt = (og + 1) % 2

            @pl.when(og > 0)
            def _():
                pltpu.make_async_copy(src_hbm.at[c, s, og], sv.at[sl], esem).wait()
                pltpu.make_async_copy(dst_hbm.at[s, og], dv.at[sl], esem).wait()
                pltpu.make_async_copy(w_hbm.at[s, og], wv.at[sl], esem).wait()

            @pl.when(og + 1 < n_super)
            def _():
                pltpu.async_copy(src_hbm.at[c, s, og + 1], sv.at[nxt], esem)
                pltpu.async_copy(dst_hbm.at[s, og + 1], dv.at[nxt], esem)
                pltpu.async_copy(w_hbm.at[s, og + 1], wv.at[nxt], esem)

            gd = {}
            sd = {}
            for i in range(D):
                gd[i] = pltpu.async_copy(xT2_hbm.at[sv.at[sl, i]],
                                         rows.at[i], gsems[i])
            for cch in range(SB):
                b = cch % D
                gd[cch].wait()
                if cch >= D:
                    sd[cch - D].wait()
                scale(sl, cch)
                sd[cch] = pltpu.async_copy(sbuf.at[b], acc_sh.at[dv.at[sl, cch]],
                                           ssems[b], add=True)
                if cch + D < SB:
                    gd[cch + D] = pltpu.async_copy(xT2_hbm.at[sv.at[sl, cch + D]],
                                                   rows.at[b], gsems[b])
            for cch in range(SB - D, SB):
                sd[cch].wait()
            return carry
        lax.fori_loop(0, n_super, sloop, 0)
        plsc.subcore_barrier()

        def dloop(g, carry):
            pltpu.sync_copy(acc_sh.at[pl.ds(row0 + g * 16, 16)],
                            out_hbm.at[c, pl.ds(row0 + g * 16, 16)])
            return carry
        lax.fori_loop(0, nz // 16, dloop, 0)

    return pl.kernel(
        body,
        out_type=jax.ShapeDtypeStruct((NC, n_out_pad, BH), jnp.float32),
        mesh=mesh,
        scratch_types=[
            pltpu.VMEM((2, SB, K), jnp.int32),
            pltpu.VMEM((2, SB, K), jnp.int32),
            pltpu.VMEM((2, SB, K), jnp.float32),
            pltpu.VMEM((D, K, BH), jnp.float32),
            pltpu.VMEM((D, K, BH), jnp.float32),
            pltpu.VMEM((16, BH), jnp.float32),
            pltpu.VMEM_SHARED((n_out_pad, BH), jnp.float32),
            [pltpu.SemaphoreType.DMA] * D,
            [pltpu.SemaphoreType.DMA] * D,
            pltpu.SemaphoreType.DMA,
        ],
    )(xT2, src_g, dst_g, w_g)


def _softplus(r):
    return jnp.log1p(jnp.exp(r))


def _kl(mu, sigma):
    return jnp.sum(-jnp.log(sigma) + 0.5 * (sigma ** 2 + mu ** 2) - 0.5)


def _pack(hT):
    n = hT.shape[0]
    hp = jnp.pad(hT, ((0, 0), (0, 2 * BH - hT.shape[1])))
    return hp.reshape(n, NC, BH).swapaxes(0, 1).reshape(NC * n, BH)


def _unpack(out2, n_out):
    return jnp.concatenate([out2[0, :n_out], out2[1, :n_out]], axis=1)[:, :200]


def _sparse_layer(hT, src, dst, wmu, wrho, bmu, brho, ew, eb, n_in, n_out):
    sw = _softplus(wrho)
    sb = _softplus(brho)
    w = wmu + sw * ew
    b = bmu + sb * eb
    kl = _kl(wmu, sw) + _kl(bmu, sb)

    E = src.shape[0]
    Ep = _rup(E, NS * SB * K)
    pad = Ep - E
    srcp = jnp.pad(src, (0, pad))
    dstp = jnp.pad(dst, (0, pad))
    wp = jnp.pad(w, (0, pad))
    n_super = Ep // (NS * SB * K)
    src_g = jnp.stack([srcp, srcp + n_in]).reshape(NC, NS, n_super, SB, K)
    dst_g = dstp.reshape(NS, n_super, SB, K)
    w_g = wp.reshape(NS, n_super, SB, K)

    n_out_pad = _rup(n_out, 256)
    out2 = _sc_segment(_pack(hT), src_g, dst_g, w_g, n_out_pad, n_super)
    hT_out = _unpack(out2, n_out) + b[:, None]
    return hT_out, kl


def _bn_t(hT, g, b):
    m = jnp.mean(hT, axis=1, keepdims=True)
    v = jnp.var(hT, axis=1, keepdims=True)
    xn = (hT - m) / jnp.sqrt(v + 1e-5)
    if g is None:
        return xn
    return xn * g[:, None] + b[:, None]


_SIZES = [10000, 2500, 625, 156, 39, 10, 3]


def kernel(x, sl0_src, sl0_dst, sl0_wmu, sl0_wrho, sl0_bmu, sl0_brho, sl0_ew, sl0_eb, sl1_src, sl1_dst, sl1_wmu, sl1_wrho, sl1_bmu, sl1_brho, sl1_ew, sl1_eb, sl2_src, sl2_dst, sl2_wmu, sl2_wrho, sl2_bmu, sl2_brho, sl2_ew, sl2_eb, sl3_src, sl3_dst, sl3_wmu, sl3_wrho, sl3_bmu, sl3_brho, sl3_ew, sl3_eb, sl4_src, sl4_dst, sl4_wmu, sl4_wrho, sl4_bmu, sl4_brho, sl4_ew, sl4_eb, sl5_src, sl5_dst, sl5_wmu, sl5_wrho, sl5_bmu, sl5_brho, sl5_ew, sl5_eb, sl6_src, sl6_dst, sl6_wmu, sl6_wrho, sl6_bmu, sl6_brho, sl6_ew, sl6_eb, sp1_src, sp1_dst, sp1_wmu, sp1_wrho, sp1_bmu, sp1_brho, sp1_ew, sp1_eb, sp2_src, sp2_dst, sp2_wmu, sp2_wrho, sp2_bmu, sp2_brho, sp2_ew, sp2_eb, sp3_src, sp3_dst, sp3_wmu, sp3_wrho, sp3_bmu, sp3_brho, sp3_ew, sp3_eb, sp4_src, sp4_dst, sp4_wmu, sp4_wrho, sp4_bmu, sp4_brho, sp4_ew, sp4_eb, sp5_src, sp5_dst, sp5_wmu, sp5_wrho, sp5_bmu, sp5_brho, sp5_ew, sp5_eb, sp6_src, sp6_dst, sp6_wmu, sp6_wrho, sp6_bmu, sp6_brho, sp6_ew, sp6_eb, bn0_g, bn0_b, bn1_g, bn1_b, bn2_g, bn2_b, bn3_g, bn3_b, bn4_g, bn4_b, bn5_g, bn5_b):
    kw = dict(locals())
    hT = x.reshape(x.shape[0], -1).T

    hT, kl_tot = _sparse_layer(hT, sl0_src, sl0_dst, sl0_wmu, sl0_wrho,
                               sl0_bmu, sl0_brho, sl0_ew, sl0_eb,
                               _SIZES[0], _SIZES[0])
    hT = jax.nn.relu(_bn_t(hT, bn0_g, bn0_b))
    for i in range(1, 7):
        hT, kl = _sparse_layer(hT, kw[f"sp{i}_src"], kw[f"sp{i}_dst"],
                               kw[f"sp{i}_wmu"], kw[f"sp{i}_wrho"],
                               kw[f"sp{i}_bmu"], kw[f"sp{i}_brho"],
                               kw[f"sp{i}_ew"], kw[f"sp{i}_eb"],
                               _SIZES[i - 1], _SIZES[i])
        kl_tot = kl_tot + kl
        hT, kl = _sparse_layer(hT, kw[f"sl{i}_src"], kw[f"sl{i}_dst"],
                               kw[f"sl{i}_wmu"], kw[f"sl{i}_wrho"],
                               kw[f"sl{i}_bmu"], kw[f"sl{i}_brho"],
                               kw[f"sl{i}_ew"], kw[f"sl{i}_eb"],
                               _SIZES[i], _SIZES[i])
        kl_tot = kl_tot + kl
        if i < 6:
            hT = jax.nn.relu(_bn_t(hT, kw[f"bn{i}_g"], kw[f"bn{i}_b"]))
        else:
            hT = _bn_t(hT, None, None)
    return hT.T, kl_tot


# speedup vs baseline: 1.3724x; 1.3724x over previous
import functools

import jax
import jax.numpy as jnp
from jax import lax
from jax.experimental import pallas as pl
from jax.experimental.pallas import tpu as pltpu
from jax.experimental.pallas import tpu_sc as plsc

L = 16
NS = 16
NC = 2
BH = 128
NB = BH // L
K = 64
SB = 4
D = 2


def _rup(a, b):
    return -(-a // b) * b


def _phase(c, s, gref, src_h, dst_h, w_h, acc, has_core, n_super,
           sv, dv, wv, rows, sbuf, gsems, ssems, esem):
    def eslice(og):
        return (src_h.at[c, s, og] if has_core else src_h.at[s, og],
                dst_h.at[s, og], w_h.at[s, og])

    s0, d0, w0 = eslice(0)
    pltpu.sync_copy(s0, sv.at[0])
    pltpu.sync_copy(d0, dv.at[0])
    pltpu.sync_copy(w0, wv.at[0])

    def scale(sl, cch):
        b = cch % D

        def hloop(h, carry):
            wrow = wv[sl, cch, pl.ds(h * L, L)]
            for e16 in range(L):
                w1 = jnp.full((L,), wrow[e16])
                e = h * L + e16
                for j in range(NB):
                    sbuf[b, e, pl.ds(j * L, L)] = rows[b, e, pl.ds(j * L, L)] * w1
            return carry
        lax.fori_loop(0, K // L, hloop, 0)

    def sloop(og, carry):
        sl = og % 2
        nxt = (og + 1) % 2

        @pl.when(og > 0)
        def _():
            sg, dg, wg = eslice(og)
            pltpu.make_async_copy(sg, sv.at[sl], esem).wait()
            pltpu.make_async_copy(dg, dv.at[sl], esem).wait()
            pltpu.make_async_copy(wg, wv.at[sl], esem).wait()

        @pl.when(og + 1 < n_super)
        def _():
            sg, dg, wg = eslice(og + 1)
            pltpu.async_copy(sg, sv.at[nxt], esem)
            pltpu.async_copy(dg, dv.at[nxt], esem)
            pltpu.async_copy(wg, wv.at[nxt], esem)

        gd = {}
        sd = {}
        for i in range(D):
            gd[i] = pltpu.async_copy(gref.at[sv.at[sl, i]], rows.at[i], gsems[i])
        for cch in range(SB):
            b = cch % D
            gd[cch].wait()
            if cch >= D:
                sd[cch - D].wait()
            scale(sl, cch)
            sd[cch] = pltpu.async_copy(sbuf.at[b], acc.at[dv.at[sl, cch]],
                                       ssems[b], add=True)
            if cch + D < SB:
                gd[cch + D] = pltpu.async_copy(gref.at[sv.at[sl, cch + D]],
                                               rows.at[b], gsems[b])
        for cch in range(SB - D, SB):
            sd[cch].wait()
        return carry
    lax.fori_loop(0, n_super, sloop, 0)


def _zero_rows(acc, row0, nrows, zb_vm):
    def zloop(g, carry):
        pltpu.sync_copy(zb_vm, acc.at[pl.ds(row0 + g * 16, 16)])
        return carry
    lax.fori_loop(0, nrows // 16, zloop, 0)


@functools.partial(jax.jit, static_argnums=(4, 5))
def _sc_segment(xT2, src_g, dst_g, w_g, n_out_pad, n_super):
    mesh = plsc.VectorSubcoreMesh(core_axis_name="c", subcore_axis_name="s")
    nz = n_out_pad // NS

    def body(xT2_hbm, src_hbm, dst_hbm, w_hbm, out_hbm,
             sv, dv, wv, rows, sbuf, zb_vm, acc_sh, gsems, ssems, esem):
        c = lax.axis_index("c")
        s = lax.axis_index("s")
        row0 = s * nz
        zv = jnp.zeros((L,), jnp.float32)
        for r in range(16):
            for j in range(NB):
                zb_vm[r, pl.ds(j * L, L)] = zv
        _zero_rows(acc_sh, row0, nz, zb_vm)
        plsc.subcore_barrier()
        _phase(c, s, xT2_hbm, src_hbm, dst_hbm, w_hbm, acc_sh, True, n_super,
               sv, dv, wv, rows, sbuf, gsems, ssems, esem)
        plsc.subcore_barrier()

        def dloop(g, carry):
            pltpu.sync_copy(acc_sh.at[pl.ds(row0 + g * 16, 16)],
                            out_hbm.at[c, pl.ds(row0 + g * 16, 16)])
            return carry
        lax.fori_loop(0, nz // 16, dloop, 0)

    return pl.kernel(
        body,
        out_type=jax.ShapeDtypeStruct((NC, n_out_pad, BH), jnp.float32),
        mesh=mesh,
        scratch_types=[
            pltpu.VMEM((2, SB, K), jnp.int32),
            pltpu.VMEM((2, SB, K), jnp.int32),
            pltpu.VMEM((2, SB, K), jnp.float32),
            pltpu.VMEM((D, K, BH), jnp.float32),
            pltpu.VMEM((D, K, BH), jnp.float32),
            pltpu.VMEM((16, BH), jnp.float32),
            pltpu.VMEM_SHARED((n_out_pad, BH), jnp.float32),
            [pltpu.SemaphoreType.DMA] * D,
            [pltpu.SemaphoreType.DMA] * D,
            pltpu.SemaphoreType.DMA,
        ],
    )(xT2, src_g, dst_g, w_g)


@functools.partial(jax.jit, static_argnums=(8, 9, 10, 11))
def _sc_pair(xT2, asrc, adst, aw, ab_g, bsrc, bdst, bw,
             n_mid_pad, n_out_pad, nsup_a, nsup_b):
    mesh = plsc.VectorSubcoreMesh(core_axis_name="c", subcore_axis_name="s")
    nzm = n_mid_pad // NS
    nz = n_out_pad // NS

    def body(xT2_hbm, asrc_h, adst_h, aw_h, ab_h, bsrc_h, bdst_h, bw_h, out_hbm,
             sv, dv, wv, rows, sbuf, zb_vm, bias_vm, acc1_sh, acc2_sh,
             gsems, ssems, esem):
        c = lax.axis_index("c")
        s = lax.axis_index("s")
        zv = jnp.zeros((L,), jnp.float32)
        for r in range(16):
            for j in range(NB):
                zb_vm[r, pl.ds(j * L, L)] = zv
        _zero_rows(acc1_sh, s * nzm, nzm, zb_vm)
        _zero_rows(acc2_sh, s * nz, nz, zb_vm)
        plsc.subcore_barrier()
        _phase(c, s, xT2_hbm, asrc_h, adst_h, aw_h, acc1_sh, True, nsup_a,
               sv, dv, wv, rows, sbuf, gsems, ssems, esem)
        plsc.subcore_barrier()
        pltpu.sync_copy(ab_h.at[s], bias_vm)

        def bloop(blk, carry):
            r0 = s * nzm + blk * 16
            pltpu.sync_copy(acc1_sh.at[pl.ds(r0, 16)], rows.at[0, pl.ds(0, 16)])
            bb = bias_vm[pl.ds(blk * 16, 16)]
            for r16 in range(16):
                w1 = jnp.full((L,), bb[r16])
                for j in range(NB):
                    rows[0, r16, pl.ds(j * L, L)] = rows[0, r16, pl.ds(j * L, L)] + w1
            pltpu.sync_copy(rows.at[0, pl.ds(0, 16)], acc1_sh.at[pl.ds(r0, 16)])
            return carry
        lax.fori_loop(0, nzm // 16, bloop, 0)
        plsc.subcore_barrier()
        _phase(c, s, acc1_sh, bsrc_h, bdst_h, bw_h, acc2_sh, False, nsup_b,
               sv, dv, wv, rows, sbuf, gsems, ssems, esem)
        plsc.subcore_barrier()

        def dloop(g, carry):
            pltpu.sync_copy(acc2_sh.at[pl.ds(s * nz + g * 16, 16)],
                            out_hbm.at[c, pl.ds(s * nz + g * 16, 16)])
            return carry
        lax.fori_loop(0, nz // 16, dloop, 0)

    return pl.kernel(
        body,
        out_type=jax.ShapeDtypeStruct((NC, n_out_pad, BH), jnp.float32),
        mesh=mesh,
        scratch_types=[
            pltpu.VMEM((2, SB, K), jnp.int32),
            pltpu.VMEM((2, SB, K), jnp.int32),
            pltpu.VMEM((2, SB, K), jnp.float32),
            pltpu.VMEM((D, K, BH), jnp.float32),
            pltpu.VMEM((D, K, BH), jnp.float32),
            pltpu.VMEM((16, BH), jnp.float32),
            pltpu.VMEM((n_mid_pad // NS,), jnp.float32),
            pltpu.VMEM_SHARED((n_mid_pad, BH), jnp.float32),
            pltpu.VMEM_SHARED((n_out_pad, BH), jnp.float32),
            [pltpu.SemaphoreType.DMA] * D,
            [pltpu.SemaphoreType.DMA] * D,
            pltpu.SemaphoreType.DMA,
        ],
    )(xT2, asrc, adst, aw, ab_g, bsrc, bdst, bw)


def _softplus(r):
    return jnp.log1p(jnp.exp(r))


def _kl(mu, sigma):
    return jnp.sum(-jnp.log(sigma) + 0.5 * (sigma ** 2 + mu ** 2) - 0.5)


def _pack(hT):
    n = hT.shape[0]
    hp = jnp.pad(hT, ((0, 0), (0, 2 * BH - hT.shape[1])))
    return hp.reshape(n, NC, BH).swapaxes(0, 1).reshape(NC * n, BH)


def _unpack(out2, n_out):
    return jnp.concatenate([out2[0, :n_out], out2[1, :n_out]], axis=1)[:, :200]


def _edges(src, dst, w, n_in, with_core):
    E = src.shape[0]
    Ep = _rup(E, NS * SB * K)
    pad = Ep - E
    srcp = jnp.pad(src, (0, pad))
    dstp = jnp.pad(dst, (0, pad))
    wp = jnp.pad(w, (0, pad))
    n_super = Ep // (NS * SB * K)
    if with_core:
        src_g = jnp.stack([srcp, srcp + n_in]).reshape(NC, NS, n_super, SB, K)
    else:
        src_g = srcp.reshape(NS, n_super, SB, K)
    dst_g = dstp.reshape(NS, n_super, SB, K)
    w_g = wp.reshape(NS, n_super, SB, K)
    return src_g, dst_g, w_g, n_super


def _bayes(wmu, wrho, bmu, brho, ew, eb):
    sw = _softplus(wrho)
    sb = _softplus(brho)
    w = wmu + sw * ew
    b = bmu + sb * eb
    kl = _kl(wmu, sw) + _kl(bmu, sb)
    return w, b, kl


def _sparse_layer(hT, src, dst, wmu, wrho, bmu, brho, ew, eb, n_in, n_out):
    w, b, kl = _bayes(wmu, wrho, bmu, brho, ew, eb)
    src_g, dst_g, w_g, n_super = _edges(src, dst, w, n_in, True)
    n_out_pad = _rup(n_out, 256)
    out2 = _sc_segment(_pack(hT), src_g, dst_g, w_g, n_out_pad, n_super)
    return _unpack(out2, n_out) + b[:, None], kl


def _pair_layer(hT, asrc, adst, aP, bsrc, bdst, bP, n_in, n_mid, n_out):
    wa, ba, kla = _bayes(*aP)
    wb, bb, klb = _bayes(*bP)
    asrc_g, adst_g, aw_g, nsup_a = _edges(asrc, adst, wa, n_in, True)
    bsrc_g, bdst_g, bw_g, nsup_b = _edges(bsrc, bdst, wb, n_mid, False)
    n_mid_pad = _rup(n_mid, 256)
    n_out_pad = _rup(n_out, 256)
    ab_g = jnp.pad(ba, (0, n_mid_pad - n_mid)).reshape(NS, n_mid_pad // NS)
    out2 = _sc_pair(_pack(hT), asrc_g, adst_g, aw_g, ab_g,
                    bsrc_g, bdst_g, bw_g,
                    n_mid_pad, n_out_pad, nsup_a, nsup_b)
    return _unpack(out2, n_out) + bb[:, None], kla + klb


def _bn_t(hT, g, b):
    m = jnp.mean(hT, axis=1, keepdims=True)
    v = jnp.var(hT, axis=1, keepdims=True)
    xn = (hT - m) / jnp.sqrt(v + 1e-5)
    if g is None:
        return xn
    return xn * g[:, None] + b[:, None]


_SIZES = [10000, 2500, 625, 156, 39, 10, 3]


def kernel(x, sl0_src, sl0_dst, sl0_wmu, sl0_wrho, sl0_bmu, sl0_brho, sl0_ew, sl0_eb, sl1_src, sl1_dst, sl1_wmu, sl1_wrho, sl1_bmu, sl1_brho, sl1_ew, sl1_eb, sl2_src, sl2_dst, sl2_wmu, sl2_wrho, sl2_bmu, sl2_brho, sl2_ew, sl2_eb, sl3_src, sl3_dst, sl3_wmu, sl3_wrho, sl3_bmu, sl3_brho, sl3_ew, sl3_eb, sl4_src, sl4_dst, sl4_wmu, sl4_wrho, sl4_bmu, sl4_brho, sl4_ew, sl4_eb, sl5_src, sl5_dst, sl5_wmu, sl5_wrho, sl5_bmu, sl5_brho, sl5_ew, sl5_eb, sl6_src, sl6_dst, sl6_wmu, sl6_wrho, sl6_bmu, sl6_brho, sl6_ew, sl6_eb, sp1_src, sp1_dst, sp1_wmu, sp1_wrho, sp1_bmu, sp1_brho, sp1_ew, sp1_eb, sp2_src, sp2_dst, sp2_wmu, sp2_wrho, sp2_bmu, sp2_brho, sp2_ew, sp2_eb, sp3_src, sp3_dst, sp3_wmu, sp3_wrho, sp3_bmu, sp3_brho, sp3_ew, sp3_eb, sp4_src, sp4_dst, sp4_wmu, sp4_wrho, sp4_bmu, sp4_brho, sp4_ew, sp4_eb, sp5_src, sp5_dst, sp5_wmu, sp5_wrho, sp5_bmu, sp5_brho, sp5_ew, sp5_eb, sp6_src, sp6_dst, sp6_wmu, sp6_wrho, sp6_bmu, sp6_brho, sp6_ew, sp6_eb, bn0_g, bn0_b, bn1_g, bn1_b, bn2_g, bn2_b, bn3_g, bn3_b, bn4_g, bn4_b, bn5_g, bn5_b):
    kw = dict(locals())
    hT = x.reshape(x.shape[0], -1).T

    hT, kl_tot = _sparse_layer(hT, sl0_src, sl0_dst, sl0_wmu, sl0_wrho,
                               sl0_bmu, sl0_brho, sl0_ew, sl0_eb,
                               _SIZES[0], _SIZES[0])
    hT = jax.nn.relu(_bn_t(hT, bn0_g, bn0_b))
    for i in range(1, 7):
        aP = tuple(kw[f"sp{i}_{k}"] for k in ("wmu", "wrho", "bmu", "brho", "ew", "eb"))
        bP = tuple(kw[f"sl{i}_{k}"] for k in ("wmu", "wrho", "bmu", "brho", "ew", "eb"))
        hT, kl = _pair_layer(hT, kw[f"sp{i}_src"], kw[f"sp{i}_dst"], aP,
                             kw[f"sl{i}_src"], kw[f"sl{i}_dst"], bP,
                             _SIZES[i - 1], _SIZES[i], _SIZES[i])
        kl_tot = kl_tot + kl
        if i < 6:
            hT = jax.nn.relu(_bn_t(hT, kw[f"bn{i}_g"], kw[f"bn{i}_b"]))
        else:
            hT = _bn_t(hT, None, None)
    return hT.T, kl_tot


# speedup vs baseline: 1.4940x; 1.0886x over previous
import functools

import jax
import jax.numpy as jnp
from jax import lax
from jax.experimental import pallas as pl
from jax.experimental.pallas import tpu as pltpu
from jax.experimental.pallas import tpu_sc as plsc

L = 16
NS = 16
NC = 2
BH = 128
NB = BH // L
K = 64
SB = 4


def _rup(a, b):
    return -(-a // b) * b


def _phase(c, s, gref, src_h, dst_h, w_h, acc, has_core, n_super,
           sv, dv, wv, rows, gsems, ssems, esem):
    def eslice(og):
        return (src_h.at[c, s, og] if has_core else src_h.at[s, og],
                dst_h.at[s, og], w_h.at[s, og])

    s0, d0, w0 = eslice(0)
    pltpu.sync_copy(s0, sv.at[0])
    pltpu.sync_copy(d0, dv.at[0])
    pltpu.sync_copy(w0, wv.at[0])

    def scale(sl, cch):
        def hloop(h, carry):
            wrow = wv[sl, cch, pl.ds(h * L, L)]
            for e16 in range(L):
                w1 = jnp.full((L,), wrow[e16])
                e = h * L + e16
                for j in range(NB):
                    rows[cch, e, pl.ds(j * L, L)] = rows[cch, e, pl.ds(j * L, L)] * w1
            return carry
        lax.fori_loop(0, K // L, hloop, 0)

    def sloop(og, carry):
        sl = og % 2
        nxt = (og + 1) % 2

        @pl.when(og > 0)
        def _():
            sg, dg, wg = eslice(og)
            pltpu.make_async_copy(sg, sv.at[sl], esem).wait()
            pltpu.make_async_copy(dg, dv.at[sl], esem).wait()
            pltpu.make_async_copy(wg, wv.at[sl], esem).wait()

        @pl.when(og + 1 < n_super)
        def _():
            sg, dg, wg = eslice(og + 1)
            pltpu.async_copy(sg, sv.at[nxt], esem)
            pltpu.async_copy(dg, dv.at[nxt], esem)
            pltpu.async_copy(wg, wv.at[nxt], esem)

        gd = {}
        sd = {}
        for i in range(SB):
            gd[i] = pltpu.async_copy(gref.at[sv.at[sl, i]], rows.at[i], gsems[i])
        for cch in range(SB):
            gd[cch].wait()
            scale(sl, cch)
            sd[cch] = pltpu.async_copy(rows.at[cch], acc.at[dv.at[sl, cch]],
                                       ssems[cch], add=True)
        for cch in range(SB):
            sd[cch].wait()
        return carry
    lax.fori_loop(0, n_super, sloop, 0)


def _zero_rows(acc, row0, nrows, zb_vm):
    def zloop(g, carry):
        pltpu.sync_copy(zb_vm, acc.at[pl.ds(row0 + g * 16, 16)])
        return carry
    lax.fori_loop(0, nrows // 16, zloop, 0)


@functools.partial(jax.jit, static_argnums=(4, 5))
def _sc_segment(xT2, src_g, dst_g, w_g, n_out_pad, n_super):
    mesh = plsc.VectorSubcoreMesh(core_axis_name="c", subcore_axis_name="s")
    nz = n_out_pad // NS

    def body(xT2_hbm, src_hbm, dst_hbm, w_hbm, out_hbm,
             sv, dv, wv, rows, zb_vm, acc_sh, gsems, ssems, esem):
        c = lax.axis_index("c")
        s = lax.axis_index("s")
        row0 = s * nz
        zv = jnp.zeros((L,), jnp.float32)
        for r in range(16):
            for j in range(NB):
                zb_vm[r, pl.ds(j * L, L)] = zv
        _zero_rows(acc_sh, row0, nz, zb_vm)
        plsc.subcore_barrier()
        _phase(c, s, xT2_hbm, src_hbm, dst_hbm, w_hbm, acc_sh, True, n_super,
               sv, dv, wv, rows, gsems, ssems, esem)
        plsc.subcore_barrier()

        def dloop(g, carry):
            pltpu.sync_copy(acc_sh.at[pl.ds(row0 + g * 16, 16)],
                            out_hbm.at[c, pl.ds(row0 + g * 16, 16)])
            return carry
        lax.fori_loop(0, nz // 16, dloop, 0)

    return pl.kernel(
        body,
        out_type=jax.ShapeDtypeStruct((NC, n_out_pad, BH), jnp.float32),
        mesh=mesh,
        scratch_types=[
            pltpu.VMEM((2, SB, K), jnp.int32),
            pltpu.VMEM((2, SB, K), jnp.int32),
            pltpu.VMEM((2, SB, K), jnp.float32),
            pltpu.VMEM((SB, K, BH), jnp.float32),
            pltpu.VMEM((16, BH), jnp.float32),
            pltpu.VMEM_SHARED((n_out_pad, BH), jnp.float32),
            [pltpu.SemaphoreType.DMA] * SB,
            [pltpu.SemaphoreType.DMA] * SB,
            pltpu.SemaphoreType.DMA,
        ],
    )(xT2, src_g, dst_g, w_g)


@functools.partial(jax.jit, static_argnums=(8, 9, 10, 11))
def _sc_pair(xT2, asrc, adst, aw, ab_g, bsrc, bdst, bw,
             n_mid_pad, n_out_pad, nsup_a, nsup_b):
    mesh = plsc.VectorSubcoreMesh(core_axis_name="c", subcore_axis_name="s")
    nzm = n_mid_pad // NS
    nz = n_out_pad // NS

    def body(xT2_hbm, asrc_h, adst_h, aw_h, ab_h, bsrc_h, bdst_h, bw_h, out_hbm,
             sv, dv, wv, rows, zb_vm, bias_vm, acc1_sh, acc2_sh,
             gsems, ssems, esem):
        c = lax.axis_index("c")
        s = lax.axis_index("s")
        zv = jnp.zeros((L,), jnp.float32)
        for r in range(16):
            for j in range(NB):
                zb_vm[r, pl.ds(j * L, L)] = zv
        _zero_rows(acc1_sh, s * nzm, nzm, zb_vm)
        _zero_rows(acc2_sh, s * nz, nz, zb_vm)
        plsc.subcore_barrier()
        _phase(c, s, xT2_hbm, asrc_h, adst_h, aw_h, acc1_sh, True, nsup_a,
               sv, dv, wv, rows, gsems, ssems, esem)
        plsc.subcore_barrier()
        pltpu.sync_copy(ab_h.at[s], bias_vm)

        def bloop(blk, carry):
            r0 = s * nzm + blk * 16
            pltpu.sync_copy(acc1_sh.at[pl.ds(r0, 16)], rows.at[0, pl.ds(0, 16)])
            bb = bias_vm[pl.ds(blk * 16, 16)]
            for r16 in range(16):
                w1 = jnp.full((L,), bb[r16])
                for j in range(NB):
                    rows[0, r16, pl.ds(j * L, L)] = rows[0, r16, pl.ds(j * L, L)] + w1
            pltpu.sync_copy(rows.at[0, pl.ds(0, 16)], acc1_sh.at[pl.ds(r0, 16)])
            return carry
        lax.fori_loop(0, nzm // 16, bloop, 0)
        plsc.subcore_barrier()
        _phase(c, s, acc1_sh, bsrc_h, bdst_h, bw_h, acc2_sh, False, nsup_b,
               sv, dv, wv, rows, gsems, ssems, esem)
        plsc.subcore_barrier()

        def dloop(g, carry):
            pltpu.sync_copy(acc2_sh.at[pl.ds(s * nz + g * 16, 16)],
                            out_hbm.at[c, pl.ds(s * nz + g * 16, 16)])
            return carry
        lax.fori_loop(0, nz // 16, dloop, 0)

    return pl.kernel(
        body,
        out_type=jax.ShapeDtypeStruct((NC, n_out_pad, BH), jnp.float32),
        mesh=mesh,
        scratch_types=[
            pltpu.VMEM((2, SB, K), jnp.int32),
            pltpu.VMEM((2, SB, K), jnp.int32),
            pltpu.VMEM((2, SB, K), jnp.float32),
            pltpu.VMEM((SB, K, BH), jnp.float32),
            pltpu.VMEM((16, BH), jnp.float32),
            pltpu.VMEM((n_mid_pad // NS,), jnp.float32),
            pltpu.VMEM_SHARED((n_mid_pad, BH), jnp.float32),
            pltpu.VMEM_SHARED((n_out_pad, BH), jnp.float32),
            [pltpu.SemaphoreType.DMA] * SB,
            [pltpu.SemaphoreType.DMA] * SB,
            pltpu.SemaphoreType.DMA,
        ],
    )(xT2, asrc, adst, aw, ab_g, bsrc, bdst, bw)


def _softplus(r):
    return jnp.log1p(jnp.exp(r))


def _kl(mu, sigma):
    return jnp.sum(-jnp.log(sigma) + 0.5 * (sigma ** 2 + mu ** 2) - 0.5)


def _pack(hT):
    n = hT.shape[0]
    hp = jnp.pad(hT, ((0, 0), (0, 2 * BH - hT.shape[1])))
    return hp.reshape(n, NC, BH).swapaxes(0, 1).reshape(NC * n, BH)


def _unpack(out2, n_out):
    return jnp.concatenate([out2[0, :n_out], out2[1, :n_out]], axis=1)[:, :200]


def _edges(src, dst, w, n_in, with_core):
    E = src.shape[0]
    Ep = _rup(E, NS * SB * K)
    pad = Ep - E
    srcp = jnp.pad(src, (0, pad))
    dstp = jnp.pad(dst, (0, pad))
    wp = jnp.pad(w, (0, pad))
    n_super = Ep // (NS * SB * K)
    if with_core:
        src_g = jnp.stack([srcp, srcp + n_in]).reshape(NC, NS, n_super, SB, K)
    else:
        src_g = srcp.reshape(NS, n_super, SB, K)
    dst_g = dstp.reshape(NS, n_super, SB, K)
    w_g = wp.reshape(NS, n_super, SB, K)
    return src_g, dst_g, w_g, n_super


def _bayes(wmu, wrho, bmu, brho, ew, eb):
    sw = _softplus(wrho)
    sb = _softplus(brho)
    w = wmu + sw * ew
    b = bmu + sb * eb
    kl = _kl(wmu, sw) + _kl(bmu, sb)
    return w, b, kl


def _sparse_layer(hT, src, dst, wmu, wrho, bmu, brho, ew, eb, n_in, n_out):
    w, b, kl = _bayes(wmu, wrho, bmu, brho, ew, eb)
    src_g, dst_g, w_g, n_super = _edges(src, dst, w, n_in, True)
    n_out_pad = _rup(n_out, 256)
    out2 = _sc_segment(_pack(hT), src_g, dst_g, w_g, n_out_pad, n_super)
    return _unpack(out2, n_out) + b[:, None], kl


def _pair_layer(hT, asrc, adst, aP, bsrc, bdst, bP, n_in, n_mid, n_out):
    wa, ba, kla = _bayes(*aP)
    wb, bb, klb = _bayes(*bP)
    asrc_g, adst_g, aw_g, nsup_a = _edges(asrc, adst, wa, n_in, True)
    bsrc_g, bdst_g, bw_g, nsup_b = _edges(bsrc, bdst, wb, n_mid, False)
    n_mid_pad = _rup(n_mid, 256)
    n_out_pad = _rup(n_out, 256)
    ab_g = jnp.pad(ba, (0, n_mid_pad - n_mid)).reshape(NS, n_mid_pad // NS)
    out2 = _sc_pair(_pack(hT), asrc_g, adst_g, aw_g, ab_g,
                    bsrc_g, bdst_g, bw_g,
                    n_mid_pad, n_out_pad, nsup_a, nsup_b)
    return _unpack(out2, n_out) + bb[:, None], kla + klb


def _bn_t(hT, g, b):
    m = jnp.mean(hT, axis=1, keepdims=True)
    v = jnp.var(hT, axis=1, keepdims=True)
    xn = (hT - m) / jnp.sqrt(v + 1e-5)
    if g is None:
        return xn
    return xn * g[:, None] + b[:, None]


_SIZES = [10000, 2500, 625, 156, 39, 10, 3]


def kernel(x, sl0_src, sl0_dst, sl0_wmu, sl0_wrho, sl0_bmu, sl0_brho, sl0_ew, sl0_eb, sl1_src, sl1_dst, sl1_wmu, sl1_wrho, sl1_bmu, sl1_brho, sl1_ew, sl1_eb, sl2_src, sl2_dst, sl2_wmu, sl2_wrho, sl2_bmu, sl2_brho, sl2_ew, sl2_eb, sl3_src, sl3_dst, sl3_wmu, sl3_wrho, sl3_bmu, sl3_brho, sl3_ew, sl3_eb, sl4_src, sl4_dst, sl4_wmu, sl4_wrho, sl4_bmu, sl4_brho, sl4_ew, sl4_eb, sl5_src, sl5_dst, sl5_wmu, sl5_wrho, sl5_bmu, sl5_brho, sl5_ew, sl5_eb, sl6_src, sl6_dst, sl6_wmu, sl6_wrho, sl6_bmu, sl6_brho, sl6_ew, sl6_eb, sp1_src, sp1_dst, sp1_wmu, sp1_wrho, sp1_bmu, sp1_brho, sp1_ew, sp1_eb, sp2_src, sp2_dst, sp2_wmu, sp2_wrho, sp2_bmu, sp2_brho, sp2_ew, sp2_eb, sp3_src, sp3_dst, sp3_wmu, sp3_wrho, sp3_bmu, sp3_brho, sp3_ew, sp3_eb, sp4_src, sp4_dst, sp4_wmu, sp4_wrho, sp4_bmu, sp4_brho, sp4_ew, sp4_eb, sp5_src, sp5_dst, sp5_wmu, sp5_wrho, sp5_bmu, sp5_brho, sp5_ew, sp5_eb, sp6_src, sp6_dst, sp6_wmu, sp6_wrho, sp6_bmu, sp6_brho, sp6_ew, sp6_eb, bn0_g, bn0_b, bn1_g, bn1_b, bn2_g, bn2_b, bn3_g, bn3_b, bn4_g, bn4_b, bn5_g, bn5_b):
    kw = dict(locals())
    hT = x.reshape(x.shape[0], -1).T

    hT, kl_tot = _sparse_layer(hT, sl0_src, sl0_dst, sl0_wmu, sl0_wrho,
                               sl0_bmu, sl0_brho, sl0_ew, sl0_eb,
                               _SIZES[0], _SIZES[0])
    hT = jax.nn.relu(_bn_t(hT, bn0_g, bn0_b))
    for i in range(1, 7):
        aP = tuple(kw[f"sp{i}_{k}"] for k in ("wmu", "wrho", "bmu", "brho", "ew", "eb"))
        bP = tuple(kw[f"sl{i}_{k}"] for k in ("wmu", "wrho", "bmu", "brho", "ew", "eb"))
        hT, kl = _pair_layer(hT, kw[f"sp{i}_src"], kw[f"sp{i}_dst"], aP,
                             kw[f"sl{i}_src"], kw[f"sl{i}_dst"], bP,
                             _SIZES[i - 1], _SIZES[i], _SIZES[i])
        kl_tot = kl_tot + kl
        if i < 6:
            hT = jax.nn.relu(_bn_t(hT, kw[f"bn{i}_g"], kw[f"bn{i}_b"]))
        else:
            hT = _bn_t(hT, None, None)
    return hT.T, kl_tot


# speedup vs baseline: 2.1549x; 1.4423x over previous
import functools

import jax
import jax.numpy as jnp
from jax import lax
from jax.experimental import pallas as pl
from jax.experimental.pallas import tpu as pltpu
from jax.experimental.pallas import tpu_sc as plsc

L = 16
NS = 16
NC = 2
BH = 128
NB = BH // L
K = 32
SB = 3


def _rup(a, b):
    return -(-a // b) * b


def _phase(c, s, gref, src_h, dst_h, w_h, acc, has_core, n_super,
           sv, dv, wv, rows, gsems, ssems, esem):
    def eslice(og):
        return (src_h.at[c, s, og] if has_core else src_h.at[s, og],
                dst_h.at[s, og], w_h.at[s, og])

    s0, d0, w0 = eslice(0)
    pltpu.sync_copy(s0, sv.at[0])
    pltpu.sync_copy(d0, dv.at[0])
    pltpu.sync_copy(w0, wv.at[0])

    def scale(sl, cch):
        def eloop(e, carry):
            w1 = wv[sl, cch, e]
            for j in range(NB):
                rows[cch, e, pl.ds(j * L, L)] = rows[cch, e, pl.ds(j * L, L)] * w1
            return carry
        lax.fori_loop(0, K, eloop, 0)

    def sloop(og, carry):
        sl = og % 2
        nxt = (og + 1) % 2

        @pl.when(og > 0)
        def _():
            sg, dg, wg = eslice(og)
            pltpu.make_async_copy(sg, sv.at[sl], esem).wait()
            pltpu.make_async_copy(dg, dv.at[sl], esem).wait()
            pltpu.make_async_copy(wg, wv.at[sl], esem).wait()

        @pl.when(og + 1 < n_super)
        def _():
            sg, dg, wg = eslice(og + 1)
            pltpu.async_copy(sg, sv.at[nxt], esem)
            pltpu.async_copy(dg, dv.at[nxt], esem)
            pltpu.async_copy(wg, wv.at[nxt], esem)

        gd = {}
        sd = {}
        for i in range(SB):
            gd[i] = pltpu.async_copy(gref.at[sv.at[sl, i]], rows.at[i], gsems[i])
        for cch in range(SB):
            gd[cch].wait()
            scale(sl, cch)
            sd[cch] = pltpu.async_copy(rows.at[cch], acc.at[dv.at[sl, cch]],
                                       ssems[cch], add=True)
        for cch in range(SB):
            sd[cch].wait()
        return carry
    lax.fori_loop(0, n_super, sloop, 0)


def _zero_rows(acc, row0, nrows, zb_vm):
    def zloop(g, carry):
        pltpu.sync_copy(zb_vm, acc.at[pl.ds(row0 + g * 16, 16)])
        return carry
    lax.fori_loop(0, nrows // 16, zloop, 0)


@functools.partial(jax.jit, static_argnums=(4, 5))
def _sc_segment(xT2, src_g, dst_g, w_g, n_out_pad, n_super):
    mesh = plsc.VectorSubcoreMesh(core_axis_name="c", subcore_axis_name="s")
    nz = n_out_pad // NS

    def body(xT2_hbm, src_hbm, dst_hbm, w_hbm, out_hbm,
             sv, dv, wv, rows, zb_vm, acc_sh, gsems, ssems, esem):
        c = lax.axis_index("c")
        s = lax.axis_index("s")
        row0 = s * nz
        zv = jnp.zeros((L,), jnp.float32)
        for r in range(16):
            for j in range(NB):
                zb_vm[r, pl.ds(j * L, L)] = zv
        _zero_rows(acc_sh, row0, nz, zb_vm)
        plsc.subcore_barrier()
        _phase(c, s, xT2_hbm, src_hbm, dst_hbm, w_hbm, acc_sh, True, n_super,
               sv, dv, wv, rows, gsems, ssems, esem)
        plsc.subcore_barrier()

        def dloop(g, carry):
            pltpu.sync_copy(acc_sh.at[pl.ds(row0 + g * 16, 16)],
                            out_hbm.at[c, pl.ds(row0 + g * 16, 16)])
            return carry
        lax.fori_loop(0, nz // 16, dloop, 0)

    return pl.kernel(
        body,
        out_type=jax.ShapeDtypeStruct((NC, n_out_pad, BH), jnp.float32),
        mesh=mesh,
        scratch_types=[
            pltpu.VMEM((2, SB, K), jnp.int32),
            pltpu.VMEM((2, SB, K), jnp.int32),
            pltpu.VMEM((2, SB, K, L), jnp.float32),
            pltpu.VMEM((SB, K, BH), jnp.float32),
            pltpu.VMEM((16, BH), jnp.float32),
            pltpu.VMEM_SHARED((n_out_pad, BH), jnp.float32),
            [pltpu.SemaphoreType.DMA] * SB,
            [pltpu.SemaphoreType.DMA] * SB,
            pltpu.SemaphoreType.DMA,
        ],
    )(xT2, src_g, dst_g, w_g)


@functools.partial(jax.jit, static_argnums=(8, 9, 10, 11))
def _sc_pair(xT2, asrc, adst, aw, ab_g, bsrc, bdst, bw,
             n_mid_pad, n_out_pad, nsup_a, nsup_b):
    mesh = plsc.VectorSubcoreMesh(core_axis_name="c", subcore_axis_name="s")
    nzm = n_mid_pad // NS
    nz = n_out_pad // NS

    def body(xT2_hbm, asrc_h, adst_h, aw_h, ab_h, bsrc_h, bdst_h, bw_h, out_hbm,
             sv, dv, wv, rows, zb_vm, bias_vm, acc1_sh, acc2_sh,
             gsems, ssems, esem):
        c = lax.axis_index("c")
        s = lax.axis_index("s")
        zv = jnp.zeros((L,), jnp.float32)
        for r in range(16):
            for j in range(NB):
                zb_vm[r, pl.ds(j * L, L)] = zv
        _zero_rows(acc1_sh, s * nzm, nzm, zb_vm)
        _zero_rows(acc2_sh, s * nz, nz, zb_vm)
        plsc.subcore_barrier()
        _phase(c, s, xT2_hbm, asrc_h, adst_h, aw_h, acc1_sh, True, nsup_a,
               sv, dv, wv, rows, gsems, ssems, esem)
        plsc.subcore_barrier()
        pltpu.sync_copy(ab_h.at[s], bias_vm)

        def bloop(blk, carry):
            r0 = s * nzm + blk * 16
            pltpu.sync_copy(acc1_sh.at[pl.ds(r0, 16)], rows.at[0, pl.ds(0, 16)])

            def rloop(r, carry2):
                b1 = bias_vm[blk * 16 + r]
                for j in range(NB):
                    rows[0, r, pl.ds(j * L, L)] = rows[0, r, pl.ds(j * L, L)] + b1
                return carry2
            lax.fori_loop(0, 16, rloop, 0)
            pltpu.sync_copy(rows.at[0, pl.ds(0, 16)], acc1_sh.at[pl.ds(r0, 16)])
            return carry
        lax.fori_loop(0, nzm // 16, bloop, 0)
        plsc.subcore_barrier()
        _phase(c, s, acc1_sh, bsrc_h, bdst_h, bw_h, acc2_sh, False, nsup_b,
               sv, dv, wv, rows, gsems, ssems, esem)
        plsc.subcore_barrier()

        def dloop(g, carry):
            pltpu.sync_copy(acc2_sh.at[pl.ds(s * nz + g * 16, 16)],
                            out_hbm.at[c, pl.ds(s * nz + g * 16, 16)])
            return carry
        lax.fori_loop(0, nz // 16, dloop, 0)

    return pl.kernel(
        body,
        out_type=jax.ShapeDtypeStruct((NC, n_out_pad, BH), jnp.float32),
        mesh=mesh,
        scratch_types=[
            pltpu.VMEM((2, SB, K), jnp.int32),
            pltpu.VMEM((2, SB, K), jnp.int32),
            pltpu.VMEM((2, SB, K, L), jnp.float32),
            pltpu.VMEM((SB, K, BH), jnp.float32),
            pltpu.VMEM((16, BH), jnp.float32),
            pltpu.VMEM((n_mid_pad // NS, L), jnp.float32),
            pltpu.VMEM_SHARED((n_mid_pad, BH), jnp.float32),
            pltpu.VMEM_SHARED((n_out_pad, BH), jnp.float32),
            [pltpu.SemaphoreType.DMA] * SB,
            [pltpu.SemaphoreType.DMA] * SB,
            pltpu.SemaphoreType.DMA,
        ],
    )(xT2, asrc, adst, aw, ab_g, bsrc, bdst, bw)


def _softplus(r):
    return jnp.log1p(jnp.exp(r))


def _kl(mu, sigma):
    return jnp.sum(-jnp.log(sigma) + 0.5 * (sigma ** 2 + mu ** 2) - 0.5)


def _pack(hT):
    n = hT.shape[0]
    hp = jnp.pad(hT, ((0, 0), (0, 2 * BH - hT.shape[1])))
    return hp.reshape(n, NC, BH).swapaxes(0, 1).reshape(NC * n, BH)


def _unpack(out2, n_out):
    return jnp.concatenate([out2[0, :n_out], out2[1, :n_out]], axis=1)[:, :200]


def _edges(src, dst, w, n_in, with_core):
    E = src.shape[0]
    Ep = _rup(E, NS * SB * K)
    pad = Ep - E
    srcp = jnp.pad(src, (0, pad))
    dstp = jnp.pad(dst, (0, pad))
    wp = jnp.pad(w, (0, pad))
    n_super = Ep // (NS * SB * K)
    if with_core:
        src_g = jnp.stack([srcp, srcp + n_in]).reshape(NC, NS, n_super, SB, K)
    else:
        src_g = srcp.reshape(NS, n_super, SB, K)
    dst_g = dstp.reshape(NS, n_super, SB, K)
    w_g = jnp.broadcast_to(wp.reshape(NS, n_super, SB, K)[..., None],
                           (NS, n_super, SB, K, L))
    return src_g, dst_g, w_g, n_super


def _bayes(wmu, wrho, bmu, brho, ew, eb):
    sw = _softplus(wrho)
    sb = _softplus(brho)
    w = wmu + sw * ew
    b = bmu + sb * eb
    kl = _kl(wmu, sw) + _kl(bmu, sb)
    return w, b, kl


def _sparse_layer(hT, src, dst, wmu, wrho, bmu, brho, ew, eb, n_in, n_out):
    w, b, kl = _bayes(wmu, wrho, bmu, brho, ew, eb)
    src_g, dst_g, w_g, n_super = _edges(src, dst, w, n_in, True)
    n_out_pad = _rup(n_out, 256)
    out2 = _sc_segment(_pack(hT), src_g, dst_g, w_g, n_out_pad, n_super)
    return _unpack(out2, n_out) + b[:, None], kl


def _pair_layer(hT, asrc, adst, aP, bsrc, bdst, bP, n_in, n_mid, n_out):
    wa, ba, kla = _bayes(*aP)
    wb, bb, klb = _bayes(*bP)
    asrc_g, adst_g, aw_g, nsup_a = _edges(asrc, adst, wa, n_in, True)
    bsrc_g, bdst_g, bw_g, nsup_b = _edges(bsrc, bdst, wb, n_mid, False)
    n_mid_pad = _rup(n_mid, 256)
    n_out_pad = _rup(n_out, 256)
    ab_g = jnp.broadcast_to(
        jnp.pad(ba, (0, n_mid_pad - n_mid)).reshape(NS, n_mid_pad // NS)[..., None],
        (NS, n_mid_pad // NS, L))
    out2 = _sc_pair(_pack(hT), asrc_g, adst_g, aw_g, ab_g,
                    bsrc_g, bdst_g, bw_g,
                    n_mid_pad, n_out_pad, nsup_a, nsup_b)
    return _unpack(out2, n_out) + bb[:, None], kla + klb


def _bn_t(hT, g, b):
    m = jnp.mean(hT, axis=1, keepdims=True)
    v = jnp.var(hT, axis=1, keepdims=True)
    xn = (hT - m) / jnp.sqrt(v + 1e-5)
    if g is None:
        return xn
    return xn * g[:, None] + b[:, None]


_SIZES = [10000, 2500, 625, 156, 39, 10, 3]


def kernel(x, sl0_src, sl0_dst, sl0_wmu, sl0_wrho, sl0_bmu, sl0_brho, sl0_ew, sl0_eb, sl1_src, sl1_dst, sl1_wmu, sl1_wrho, sl1_bmu, sl1_brho, sl1_ew, sl1_eb, sl2_src, sl2_dst, sl2_wmu, sl2_wrho, sl2_bmu, sl2_brho, sl2_ew, sl2_eb, sl3_src, sl3_dst, sl3_wmu, sl3_wrho, sl3_bmu, sl3_brho, sl3_ew, sl3_eb, sl4_src, sl4_dst, sl4_wmu, sl4_wrho, sl4_bmu, sl4_brho, sl4_ew, sl4_eb, sl5_src, sl5_dst, sl5_wmu, sl5_wrho, sl5_bmu, sl5_brho, sl5_ew, sl5_eb, sl6_src, sl6_dst, sl6_wmu, sl6_wrho, sl6_bmu, sl6_brho, sl6_ew, sl6_eb, sp1_src, sp1_dst, sp1_wmu, sp1_wrho, sp1_bmu, sp1_brho, sp1_ew, sp1_eb, sp2_src, sp2_dst, sp2_wmu, sp2_wrho, sp2_bmu, sp2_brho, sp2_ew, sp2_eb, sp3_src, sp3_dst, sp3_wmu, sp3_wrho, sp3_bmu, sp3_brho, sp3_ew, sp3_eb, sp4_src, sp4_dst, sp4_wmu, sp4_wrho, sp4_bmu, sp4_brho, sp4_ew, sp4_eb, sp5_src, sp5_dst, sp5_wmu, sp5_wrho, sp5_bmu, sp5_brho, sp5_ew, sp5_eb, sp6_src, sp6_dst, sp6_wmu, sp6_wrho, sp6_bmu, sp6_brho, sp6_ew, sp6_eb, bn0_g, bn0_b, bn1_g, bn1_b, bn2_g, bn2_b, bn3_g, bn3_b, bn4_g, bn4_b, bn5_g, bn5_b):
    kw = dict(locals())
    hT = x.reshape(x.shape[0], -1).T

    hT, kl_tot = _sparse_layer(hT, sl0_src, sl0_dst, sl0_wmu, sl0_wrho,
                               sl0_bmu, sl0_brho, sl0_ew, sl0_eb,
                               _SIZES[0], _SIZES[0])
    hT = jax.nn.relu(_bn_t(hT, bn0_g, bn0_b))
    for i in range(1, 7):
        aP = tuple(kw[f"sp{i}_{k}"] for k in ("wmu", "wrho", "bmu", "brho", "ew", "eb"))
        bP = tuple(kw[f"sl{i}_{k}"] for k in ("wmu", "wrho", "bmu", "brho", "ew", "eb"))
        hT, kl = _pair_layer(hT, kw[f"sp{i}_src"], kw[f"sp{i}_dst"], aP,
                             kw[f"sl{i}_src"], kw[f"sl{i}_dst"], bP,
                             _SIZES[i - 1], _SIZES[i], _SIZES[i])
        kl_tot = kl_tot + kl
        if i < 6:
            hT = jax.nn.relu(_bn_t(hT, kw[f"bn{i}_g"], kw[f"bn{i}_b"]))
        else:
            hT = _bn_t(hT, None, None)
    return hT.T, kl_tot
